# baseline (speedup 1.0000x reference)
import functools

import jax
import jax.numpy as jnp
from jax import lax
from jax.experimental import pallas as pl
from jax.experimental.pallas import tpu as pltpu
from jax.experimental.pallas import tpu_sc as plsc

NUM_CLASSES = 100000
HD = 128
N = 16384

NC = 2
NS = 16

BLOCK = 12800
BLOCKS_PER_CORE = 4
DUMMY = BLOCK
ACC_ROWS = BLOCK + 8
LPT = N // NS
NCHUNK = LPT // 128
ROWS_PER_TILE = BLOCK // NS


def _body(labels_hbm, hv_hbm, w_hbm, out_hbm, labels_v, dst_idx, stage, acc):
    c = lax.axis_index("c")
    s = lax.axis_index("s")
    lab_base = s * LPT
    pltpu.sync_copy(labels_hbm.at[pl.ds(lab_base, LPT)], labels_v)

    for b in range(BLOCKS_PER_CORE):
        blo = (c * BLOCKS_PER_CORE + b) * BLOCK
        row0 = blo + s * ROWS_PER_TILE

        @pl.when(row0 < NUM_CLASSES)
        def _():
            pltpu.sync_copy(
                w_hbm.at[pl.ds(row0, ROWS_PER_TILE)],
                acc.at[pl.ds(s * ROWS_PER_TILE, ROWS_PER_TILE)],
            )

        plsc.subcore_barrier()

        for chunk in range(NCHUNK):
            base = chunk * 128
            pltpu.sync_copy(hv_hbm.at[pl.ds(lab_base + base, 128)], stage)
            for g in range(8):
                lab = labels_v[pl.ds(base + g * 16, 16)]
                in_blk = (lab >= blo) & (lab < blo + BLOCK)
                dst_idx[pl.ds(g * 16, 16)] = jnp.where(in_blk, lab - blo, DUMMY)
            pltpu.sync_copy(stage, acc.at[dst_idx], add=True)

        plsc.subcore_barrier()

        @pl.when(row0 < NUM_CLASSES)
        def _():
            pltpu.sync_copy(
                acc.at[pl.ds(s * ROWS_PER_TILE, ROWS_PER_TILE)],
                out_hbm.at[pl.ds(row0, ROWS_PER_TILE)],
            )

        plsc.subcore_barrier()


@jax.jit
def _scatter_add(labels, hv, classify_weights):
    mesh = plsc.VectorSubcoreMesh(
        core_axis_name="c", subcore_axis_name="s", num_cores=NC, num_subcores=NS
    )
    return pl.kernel(
        _body,
        out_type=jax.ShapeDtypeStruct((NUM_CLASSES, HD), jnp.float32),
        mesh=mesh,
        scratch_types=[
            pltpu.VMEM((LPT,), jnp.int32),
            pltpu.VMEM((128,), jnp.int32),
            pltpu.VMEM((128, HD), jnp.float32),
            pltpu.VMEM_SHARED((ACC_ROWS, HD), jnp.float32),
        ],
    )(labels, hv, classify_weights)


def kernel(labels, hv, classify_weights):
    return _scatter_add(labels, hv, classify_weights)
